# Initial kernel scaffold; baseline (speedup 1.0000x reference)
import functools

import jax
import jax.numpy as jnp
import numpy as np
from jax import lax
from jax.experimental import pallas as pl
from jax.experimental.pallas import tpu as pltpu
from jax.experimental.pallas import tpu_sc as plsc

D_MODEL = 256
NHEAD = 8
HD = 32
NLEV = 4
NPOINT = 4
DFF = 2048
SHAPES_C = [(64, 64), (32, 32), (16, 16), (8, 8)]
LQ = sum(h * w for h, w in SHAPES_C)
B = 2
T = B * LQ
NSAMP = NLEV * NPOINT
NCOL = NHEAD * NSAMP

_bases = []
_off = 1
for _h, _w in SHAPES_C:
    _bases.append(_off)
    _off += (_h + 2) * (_w + 2) + 1
BH_ROWS = _off
BH_STRIDE = 5944
N_BH = B * NHEAD
TABLE_ROWS = N_BH * BH_STRIDE

QH = B * NHEAD * LQ
NW = 32
QH_PER_W = QH // NW
CHUNK_QH = 4
N_CHUNK = QH_PER_W // CHUNK_QH

ROW_BLK = 544
N_BLK = T // ROW_BLK
BLK_PER_B = LQ // ROW_BLK


def _np_col_consts():
    w_col = np.zeros(NCOL, np.float32)
    h_col = np.zeros(NCOL, np.float32)
    base_col = np.zeros(NCOL, np.int32)
    head_col = np.zeros(NCOL, np.int32)
    for h in range(NHEAD):
        for l in range(NLEV):
            for p in range(NPOINT):
                c = h * NSAMP + l * NPOINT + p
                h_, w_ = SHAPES_C[l]
                w_col[c] = w_
                h_col[c] = h_
                base_col[c] = _bases[l]
                head_col[c] = h
    return w_col, h_col, base_col, head_col


def _np_ref_points():
    rps = []
    for (h_, w_) in SHAPES_C:
        ry, rx = np.meshgrid(np.linspace(0.5, h_ - 0.5, h_, dtype=np.float64),
                             np.linspace(0.5, w_ - 0.5, w_, dtype=np.float64),
                             indexing='ij')
        rps.append(np.stack([rx.reshape(-1) / w_, ry.reshape(-1) / h_], axis=-1))
    return np.concatenate(rps, axis=0).astype(np.float32)


_GROUP_ONES = np.kron(np.eye(NHEAD, dtype=np.float32),
                      np.ones((NSAMP, NSAMP), np.float32))


def _proj_body(src_ref, pos_ref, refx_ref, refy_ref,
               wval_ref, bval_ref, woffx_ref, woffy_ref, boffx_ref, boffy_ref,
               wattn_ref, battn_ref, gones_ref,
               wcol_ref, hcol_ref, basecol_ref, headcol_ref,
               value_ref, idx0_ref, idx1_ref, w4_ref):
    x = src_ref[...]
    q = x + pos_ref[...]
    value_ref[...] = jnp.dot(x, wval_ref[...],
                             preferred_element_type=jnp.float32) + bval_ref[...]
    offx = jnp.dot(q, woffx_ref[...], preferred_element_type=jnp.float32) + boffx_ref[...]
    offy = jnp.dot(q, woffy_ref[...], preferred_element_type=jnp.float32) + boffy_ref[...]
    logits = jnp.dot(q, wattn_ref[...], preferred_element_type=jnp.float32) + battn_ref[...]
    m = jnp.max(logits, axis=-1, keepdims=True)
    e = jnp.exp(logits - m)
    denom = jnp.dot(e, gones_ref[...], preferred_element_type=jnp.float32)
    aw = e / denom

    wcol = wcol_ref[...]
    hcol = hcol_ref[...]
    gx = refx_ref[...] * wcol + offx - 0.5
    gy = refy_ref[...] * hcol + offy - 0.5
    x0 = jnp.floor(gx)
    y0 = jnp.floor(gy)
    wx1 = gx - x0
    wy1 = gy - y0
    wx0 = 1.0 - wx1
    wy0 = 1.0 - wy1
    xp = jnp.clip(x0, -2.0, wcol).astype(jnp.int32) + 1
    yr0 = jnp.clip(y0, -1.0, hcol).astype(jnp.int32) + 1
    yr1 = jnp.clip(y0 + 1.0, -1.0, hcol).astype(jnp.int32) + 1
    wp2 = wcol.astype(jnp.int32) + 2
    b = pl.program_id(0) // BLK_PER_B
    bh_base = (b * NHEAD + headcol_ref[...]) * BH_STRIDE + basecol_ref[...]
    idx0_ref[...] = bh_base + yr0 * wp2 + xp
    idx1_ref[...] = bh_base + yr1 * wp2 + xp
    w4_ref[0] = aw * wy0 * wx0
    w4_ref[1] = aw * wy0 * wx1
    w4_ref[2] = aw * wy1 * wx0
    w4_ref[3] = aw * wy1 * wx1


def _run_proj(src2d, pos2d, refx, refy, W_val, b_val, W_offx, W_offy,
              b_offx, b_offy, W_attn, b_attn):
    wcol, hcol, basecol, headcol = _np_col_consts()
    full = lambda r, c: pl.BlockSpec((r, c), lambda i: (0, 0))
    blk = lambda c: pl.BlockSpec((ROW_BLK, c), lambda i: (i, 0))
    w4spec = pl.BlockSpec((4, ROW_BLK, NCOL), lambda i: (0, i, 0))
    return pl.pallas_call(
        _proj_body,
        grid=(N_BLK,),
        in_specs=[blk(D_MODEL), blk(D_MODEL), blk(1), blk(1),
                  full(D_MODEL, D_MODEL), full(1, D_MODEL),
                  full(D_MODEL, NCOL), full(D_MODEL, NCOL),
                  full(1, NCOL), full(1, NCOL),
                  full(D_MODEL, NCOL), full(1, NCOL), full(NCOL, NCOL),
                  full(1, NCOL), full(1, NCOL), full(1, NCOL), full(1, NCOL)],
        out_specs=[blk(D_MODEL), blk(NCOL), blk(NCOL), w4spec],
        out_shape=[jax.ShapeDtypeStruct((T, D_MODEL), jnp.float32),
                   jax.ShapeDtypeStruct((T, NCOL), jnp.int32),
                   jax.ShapeDtypeStruct((T, NCOL), jnp.int32),
                   jax.ShapeDtypeStruct((4, T, NCOL), jnp.float32)],
    )(src2d, pos2d, refx, refy, W_val, b_val.reshape(1, -1),
      W_offx, W_offy, b_offx.reshape(1, -1), b_offy.reshape(1, -1),
      W_attn, b_attn.reshape(1, -1), jnp.asarray(_GROUP_ONES),
      jnp.asarray(wcol).reshape(1, -1), jnp.asarray(hcol).reshape(1, -1),
      jnp.asarray(basecol).reshape(1, -1), jnp.asarray(headcol).reshape(1, -1))


def _build_table(value):
    v = value.reshape(B, LQ, NHEAD, HD).transpose(0, 2, 1, 3).reshape(N_BH, LQ, HD)
    rows = [jnp.zeros((N_BH, 1, HD), jnp.float32)]
    start = 0
    for (h_, w_) in SHAPES_C:
        m = v[:, start:start + h_ * w_].reshape(N_BH, h_, w_, HD)
        m = jnp.pad(m, ((0, 0), (1, 1), (1, 1), (0, 0)))
        rows.append(m.reshape(N_BH, (h_ + 2) * (w_ + 2), HD))
        rows.append(jnp.zeros((N_BH, 1, HD), jnp.float32))
        start += h_ * w_
    P = jnp.concatenate(rows, axis=1)
    P = jnp.pad(P, ((0, 0), (0, BH_STRIDE - BH_ROWS), (0, 0)))
    Pf = P.reshape(N_BH * BH_STRIDE, HD)
    Pn = jnp.concatenate([Pf[1:], jnp.zeros((1, HD), jnp.float32)], axis=0)
    return jnp.concatenate([Pf, Pn], axis=1)


def _sc_body(idx_hbm, wts_hbm, table_hbm, out_hbm, idx_v, wts_v, rows_v, out_v, sem):
    wid = lax.axis_index("s") * 2 + lax.axis_index("c")
    tile_base = wid * QH_PER_W

    def chunk(i, carry):
        base = tile_base + i * CHUNK_QH
        pltpu.sync_copy(idx_hbm.at[pl.ds(base * 2 * NSAMP, CHUNK_QH * 2 * NSAMP)], idx_v)
        pltpu.sync_copy(wts_hbm.at[pl.ds(base * 4 * NSAMP, CHUNK_QH * 4 * NSAMP)], wts_v)
        pltpu.async_copy(table_hbm.at[idx_v], rows_v, sem).wait()
        for c in range(CHUNK_QH):
            acc0 = jnp.zeros((16,), jnp.float32)
            acc1 = jnp.zeros((16,), jnp.float32)
            for s in range(2 * NSAMP):
                r = c * 2 * NSAMP + s
                wa = jnp.full((16,), wts_v[2 * r], jnp.float32)
                wb = jnp.full((16,), wts_v[2 * r + 1], jnp.float32)
                acc0 = acc0 + wa * rows_v[r, 0:16] + wb * rows_v[r, 32:48]
                acc1 = acc1 + wa * rows_v[r, 16:32] + wb * rows_v[r, 48:64]
            out_v[pl.ds(c * HD, 16)] = acc0
            out_v[pl.ds(c * HD + 16, 16)] = acc1
        pltpu.sync_copy(out_v, out_hbm.at[pl.ds(base * HD, CHUNK_QH * HD)])
        return carry

    lax.fori_loop(0, N_CHUNK, chunk, 0)


def _run_sc(idx_flat, wts_flat, table):
    mesh = plsc.VectorSubcoreMesh(core_axis_name="c", subcore_axis_name="s")
    kern = pl.kernel(
        _sc_body,
        out_type=jax.ShapeDtypeStruct((QH * HD,), jnp.float32),
        mesh=mesh,
        scratch_types=[
            pltpu.VMEM((CHUNK_QH * 2 * NSAMP,), jnp.int32),
            pltpu.VMEM((CHUNK_QH * 4 * NSAMP,), jnp.float32),
            pltpu.VMEM((CHUNK_QH * 2 * NSAMP, 2 * HD), jnp.float32),
            pltpu.VMEM((CHUNK_QH * HD,), jnp.float32),
            pltpu.SemaphoreType.DMA,
        ],
    )
    return kern(idx_flat, wts_flat, table)


def _tail_body(attn_ref, src_ref, wout_ref, bout_ref, w1_ref, b1_ref,
               w2_ref, b2_ref, g1_ref, be1_ref, g2_ref, be2_ref, out_ref):
    def ln(x, g, bta):
        mu = jnp.mean(x, axis=-1, keepdims=True)
        xc = x - mu
        var = jnp.mean(xc * xc, axis=-1, keepdims=True)
        return xc * jax.lax.rsqrt(var + 1e-5) * g + bta

    src2 = jnp.dot(attn_ref[...], wout_ref[...],
                   preferred_element_type=jnp.float32) + bout_ref[...]
    x = ln(src_ref[...] + src2, g1_ref[...], be1_ref[...])
    h = jnp.maximum(jnp.dot(x, w1_ref[...], preferred_element_type=jnp.float32)
                    + b1_ref[...], 0.0)
    ff = jnp.dot(h, w2_ref[...], preferred_element_type=jnp.float32) + b2_ref[...]
    out_ref[...] = ln(x + ff, g2_ref[...], be2_ref[...])


def _run_tail(attn2d, src2d, W_out, b_out, W1, b1, W2, b2, g1, be1, g2, be2):
    full = lambda r, c: pl.BlockSpec((r, c), lambda i: (0, 0))
    blk = lambda c: pl.BlockSpec((ROW_BLK, c), lambda i: (i, 0))
    return pl.pallas_call(
        _tail_body,
        grid=(N_BLK,),
        in_specs=[blk(D_MODEL), blk(D_MODEL),
                  full(D_MODEL, D_MODEL), full(1, D_MODEL),
                  full(D_MODEL, DFF), full(1, DFF),
                  full(DFF, D_MODEL), full(1, D_MODEL),
                  full(1, D_MODEL), full(1, D_MODEL),
                  full(1, D_MODEL), full(1, D_MODEL)],
        out_specs=blk(D_MODEL),
        out_shape=jax.ShapeDtypeStruct((T, D_MODEL), jnp.float32),
    )(attn2d, src2d, W_out, b_out.reshape(1, -1), W1, b1.reshape(1, -1),
      W2, b2.reshape(1, -1), g1.reshape(1, -1), be1.reshape(1, -1),
      g2.reshape(1, -1), be2.reshape(1, -1))


def kernel(src, spatial_shapes, level_start_index, pos,
           W_off, b_off, W_attn, b_attn, W_val, b_val, W_out, b_out,
           W1, b1, W2, b2, g1, be1, g2, be2):
    del spatial_shapes, level_start_index
    src2d = src.reshape(T, D_MODEL)
    pos2d = pos.reshape(T, D_MODEL)
    refp = jnp.asarray(_np_ref_points())
    refx = jnp.tile(refp[:, 0:1], (B, 1))
    refy = jnp.tile(refp[:, 1:2], (B, 1))
    W_off3 = W_off.reshape(D_MODEL, NCOL, 2)
    b_off2 = b_off.reshape(NCOL, 2)

    value, idx0, idx1, w4 = _run_proj(
        src2d, pos2d, refx, refy, W_val, b_val,
        W_off3[..., 0], W_off3[..., 1], b_off2[:, 0], b_off2[:, 1],
        W_attn, b_attn)

    table = _build_table(value)
    idx = jnp.stack([idx0, idx1], axis=-1)
    idx_flat = (idx.reshape(B, LQ, NHEAD, NSAMP, 2)
                .transpose(0, 2, 1, 3, 4).reshape(-1))
    wts = jnp.stack([w4[0], w4[1], w4[2], w4[3]], axis=-1)
    wts_flat = (wts.reshape(B, LQ, NHEAD, NSAMP, 4)
                .transpose(0, 2, 1, 3, 4).reshape(-1))

    attn_flat = _run_sc(idx_flat, wts_flat, table)
    attn2d = (attn_flat.reshape(B, NHEAD, LQ, HD)
              .transpose(0, 2, 1, 3).reshape(T, D_MODEL))

    out = _run_tail(attn2d, src2d, W_out, b_out, W1, b1, W2, b2,
                    g1, be1, g2, be2)
    return out.reshape(B, LQ, D_MODEL)


# speedup vs baseline: 985.7721x; 985.7721x over previous
import functools

import jax
import jax.numpy as jnp
import numpy as np
from jax import lax
from jax.experimental import pallas as pl
from jax.experimental.pallas import tpu as pltpu
from jax.experimental.pallas import tpu_sc as plsc

D_MODEL = 256
NHEAD = 8
HD = 32
NLEV = 4
NPOINT = 4
DFF = 2048
SHAPES_C = [(64, 64), (32, 32), (16, 16), (8, 8)]
LQ = sum(h * w for h, w in SHAPES_C)
B = 2
T = B * LQ
NSAMP = NLEV * NPOINT
NCOL = NHEAD * NSAMP

_bases = []
_off = 0
for _h, _w in SHAPES_C:
    _bases.append(_off)
    _off += (_h + 2) * (_w + 2)
BH_STRIDE = _off
N_BH = B * NHEAD
TABLE_ROWS = N_BH * BH_STRIDE

QH = B * NHEAD * LQ
NW = 32
QH_PER_W = QH // NW
CHUNK_QH = 8
N_CHUNK = QH_PER_W // CHUNK_QH

ROW_BLK = 544
N_BLK = T // ROW_BLK
BLK_PER_B = LQ // ROW_BLK


def _np_col_consts():
    w_col = np.zeros(NCOL, np.float32)
    h_col = np.zeros(NCOL, np.float32)
    base_col = np.zeros(NCOL, np.int32)
    head_col = np.zeros(NCOL, np.int32)
    for h in range(NHEAD):
        for l in range(NLEV):
            for p in range(NPOINT):
                c = h * NSAMP + l * NPOINT + p
                h_, w_ = SHAPES_C[l]
                w_col[c] = w_
                h_col[c] = h_
                base_col[c] = _bases[l]
                head_col[c] = h
    return w_col, h_col, base_col, head_col


def _np_ref_points():
    rps = []
    for (h_, w_) in SHAPES_C:
        ry, rx = np.meshgrid(np.linspace(0.5, h_ - 0.5, h_, dtype=np.float64),
                             np.linspace(0.5, w_ - 0.5, w_, dtype=np.float64),
                             indexing='ij')
        rps.append(np.stack([rx.reshape(-1) / w_, ry.reshape(-1) / h_], axis=-1))
    return np.concatenate(rps, axis=0).astype(np.float32)


_GROUP_ONES = np.kron(np.eye(NHEAD, dtype=np.float32),
                      np.ones((NSAMP, NSAMP), np.float32))


def _proj_body(src_ref, pos_ref, refx_ref, refy_ref,
               wval_ref, bval_ref, woffx_ref, woffy_ref, boffx_ref, boffy_ref,
               wattn_ref, battn_ref, gones_ref,
               wcol_ref, hcol_ref, basecol_ref, headcol_ref,
               value_ref, idx0_ref, w4_ref):
    x = src_ref[...]
    q = x + pos_ref[...]
    value_ref[...] = jnp.dot(x, wval_ref[...],
                             preferred_element_type=jnp.float32) + bval_ref[...]
    offx = jnp.dot(q, woffx_ref[...], preferred_element_type=jnp.float32) + boffx_ref[...]
    offy = jnp.dot(q, woffy_ref[...], preferred_element_type=jnp.float32) + boffy_ref[...]
    logits = jnp.dot(q, wattn_ref[...], preferred_element_type=jnp.float32) + battn_ref[...]
    m = jnp.max(logits, axis=-1, keepdims=True)
    e = jnp.exp(logits - m)
    denom = jnp.dot(e, gones_ref[...], preferred_element_type=jnp.float32)
    aw = e / denom

    wcol = wcol_ref[...]
    hcol = hcol_ref[...]
    gx = refx_ref[...] * wcol + offx - 0.5
    gy = refy_ref[...] * hcol + offy - 0.5
    x0 = jnp.floor(gx)
    y0 = jnp.floor(gy)
    wx1 = gx - x0
    wy1 = gy - y0
    wx0 = 1.0 - wx1
    wy0 = 1.0 - wy1
    xp = jnp.clip(x0, -1.0, wcol).astype(jnp.int32) + 1
    yr = jnp.clip(y0, -1.0, hcol).astype(jnp.int32) + 1
    msk = ((x0 >= -1.0) & (x0 <= wcol) & (y0 >= -1.0) & (y0 <= hcol)
           ).astype(jnp.float32)
    wp2 = wcol.astype(jnp.int32) + 2
    b = pl.program_id(0) // BLK_PER_B
    bh_base = (b * NHEAD + headcol_ref[...]) * BH_STRIDE + basecol_ref[...]
    idx0_ref[...] = bh_base + yr * wp2 + xp
    awm = aw * msk
    w4_ref[0] = awm * wy0 * wx0
    w4_ref[1] = awm * wy0 * wx1
    w4_ref[2] = awm * wy1 * wx0
    w4_ref[3] = awm * wy1 * wx1


def _run_proj(src2d, pos2d, refx, refy, W_val, b_val, W_offx, W_offy,
              b_offx, b_offy, W_attn, b_attn):
    wcol, hcol, basecol, headcol = _np_col_consts()
    full = lambda r, c: pl.BlockSpec((r, c), lambda i: (0, 0))
    blk = lambda c: pl.BlockSpec((ROW_BLK, c), lambda i: (i, 0))
    w4spec = pl.BlockSpec((4, ROW_BLK, NCOL), lambda i: (0, i, 0))
    return pl.pallas_call(
        _proj_body,
        grid=(N_BLK,),
        in_specs=[blk(D_MODEL), blk(D_MODEL), blk(1), blk(1),
                  full(D_MODEL, D_MODEL), full(1, D_MODEL),
                  full(D_MODEL, NCOL), full(D_MODEL, NCOL),
                  full(1, NCOL), full(1, NCOL),
                  full(D_MODEL, NCOL), full(1, NCOL), full(NCOL, NCOL),
                  full(1, NCOL), full(1, NCOL), full(1, NCOL), full(1, NCOL)],
        out_specs=[blk(D_MODEL), blk(NCOL), w4spec],
        out_shape=[jax.ShapeDtypeStruct((T, D_MODEL), jnp.float32),
                   jax.ShapeDtypeStruct((T, NCOL), jnp.int32),
                   jax.ShapeDtypeStruct((4, T, NCOL), jnp.float32)],
    )(src2d, pos2d, refx, refy, W_val, b_val.reshape(1, -1),
      W_offx, W_offy, b_offx.reshape(1, -1), b_offy.reshape(1, -1),
      W_attn, b_attn.reshape(1, -1), jnp.asarray(_GROUP_ONES),
      jnp.asarray(wcol).reshape(1, -1), jnp.asarray(hcol).reshape(1, -1),
      jnp.asarray(basecol).reshape(1, -1), jnp.asarray(headcol).reshape(1, -1))


def _build_table(value):
    v = value.reshape(B, LQ, NHEAD, HD).transpose(0, 2, 1, 3).reshape(N_BH, LQ, HD)
    segs = []
    start = 0
    for (h_, w_) in SHAPES_C:
        m = v[:, start:start + h_ * w_].reshape(N_BH, h_, w_, HD)
        m = jnp.pad(m, ((0, 0), (1, 1), (1, 1), (0, 0)))
        sx = jnp.pad(m[:, :, 1:], ((0, 0), (0, 0), (0, 1), (0, 0)))
        sy = jnp.pad(m[:, 1:], ((0, 0), (0, 1), (0, 0), (0, 0)))
        sxy = jnp.pad(m[:, 1:, 1:], ((0, 0), (0, 1), (0, 1), (0, 0)))
        patch = jnp.concatenate([m, sx, sy, sxy], axis=-1)
        segs.append(patch.reshape(N_BH, (h_ + 2) * (w_ + 2), 4 * HD))
        start += h_ * w_
    P = jnp.concatenate(segs, axis=1)
    return P.reshape(TABLE_ROWS, 4 * HD)


def _sc_body(idx_hbm, wts_hbm, table_hbm, out_hbm, idx_v, wts_v, rows_v, out_v, sem):
    wid = lax.axis_index("s") * 2 + lax.axis_index("c")
    tile_base = wid * QH_PER_W

    def chunk(i, carry):
        base = tile_base + i * CHUNK_QH
        pltpu.sync_copy(idx_hbm.at[pl.ds(base * NSAMP, CHUNK_QH * NSAMP)], idx_v)
        pltpu.sync_copy(wts_hbm.at[pl.ds(base * 4 * NSAMP, CHUNK_QH * 4 * NSAMP)], wts_v)
        pltpu.async_copy(table_hbm.at[idx_v], rows_v, sem).wait()
        for c in range(CHUNK_QH):
            acc0 = jnp.zeros((16,), jnp.float32)
            acc1 = jnp.zeros((16,), jnp.float32)
            for s in range(NSAMP):
                r = c * NSAMP + s
                wv = wts_v[pl.ds(r * 4 // 16 * 16, 16)]
                j = r * 4 % 16
                w00 = jnp.full((16,), wv[j], jnp.float32)
                w01 = jnp.full((16,), wv[j + 1], jnp.float32)
                w10 = jnp.full((16,), wv[j + 2], jnp.float32)
                w11 = jnp.full((16,), wv[j + 3], jnp.float32)
                acc0 = (acc0 + w00 * rows_v[r, 0:16] + w01 * rows_v[r, 32:48]
                        + w10 * rows_v[r, 64:80] + w11 * rows_v[r, 96:112])
                acc1 = (acc1 + w00 * rows_v[r, 16:32] + w01 * rows_v[r, 48:64]
                        + w10 * rows_v[r, 80:96] + w11 * rows_v[r, 112:128])
            out_v[pl.ds(c * HD, 16)] = acc0
            out_v[pl.ds(c * HD + 16, 16)] = acc1
        pltpu.sync_copy(out_v, out_hbm.at[pl.ds(base * HD, CHUNK_QH * HD)])
        return carry

    lax.fori_loop(0, N_CHUNK, chunk, 0)


def _run_sc(idx_flat, wts_flat, table):
    mesh = plsc.VectorSubcoreMesh(core_axis_name="c", subcore_axis_name="s")
    kern = pl.kernel(
        _sc_body,
        out_type=jax.ShapeDtypeStruct((QH * HD,), jnp.float32),
        mesh=mesh,
        scratch_types=[
            pltpu.VMEM((CHUNK_QH * NSAMP,), jnp.int32),
            pltpu.VMEM((CHUNK_QH * 4 * NSAMP,), jnp.float32),
            pltpu.VMEM((CHUNK_QH * NSAMP, 4 * HD), jnp.float32),
            pltpu.VMEM((CHUNK_QH * HD,), jnp.float32),
            pltpu.SemaphoreType.DMA,
        ],
    )
    return kern(idx_flat, wts_flat, table)


def _tail_body(attn_ref, src_ref, wout_ref, bout_ref, w1_ref, b1_ref,
               w2_ref, b2_ref, g1_ref, be1_ref, g2_ref, be2_ref, out_ref):
    def ln(x, g, bta):
        mu = jnp.mean(x, axis=-1, keepdims=True)
        xc = x - mu
        var = jnp.mean(xc * xc, axis=-1, keepdims=True)
        return xc * jax.lax.rsqrt(var + 1e-5) * g + bta

    src2 = jnp.dot(attn_ref[...], wout_ref[...],
                   preferred_element_type=jnp.float32) + bout_ref[...]
    x = ln(src_ref[...] + src2, g1_ref[...], be1_ref[...])
    h = jnp.maximum(jnp.dot(x, w1_ref[...], preferred_element_type=jnp.float32)
                    + b1_ref[...], 0.0)
    ff = jnp.dot(h, w2_ref[...], preferred_element_type=jnp.float32) + b2_ref[...]
    out_ref[...] = ln(x + ff, g2_ref[...], be2_ref[...])


def _run_tail(attn2d, src2d, W_out, b_out, W1, b1, W2, b2, g1, be1, g2, be2):
    full = lambda r, c: pl.BlockSpec((r, c), lambda i: (0, 0))
    blk = lambda c: pl.BlockSpec((ROW_BLK, c), lambda i: (i, 0))
    return pl.pallas_call(
        _tail_body,
        grid=(N_BLK,),
        in_specs=[blk(D_MODEL), blk(D_MODEL),
                  full(D_MODEL, D_MODEL), full(1, D_MODEL),
                  full(D_MODEL, DFF), full(1, DFF),
                  full(DFF, D_MODEL), full(1, D_MODEL),
                  full(1, D_MODEL), full(1, D_MODEL),
                  full(1, D_MODEL), full(1, D_MODEL)],
        out_specs=blk(D_MODEL),
        out_shape=jax.ShapeDtypeStruct((T, D_MODEL), jnp.float32),
    )(attn2d, src2d, W_out, b_out.reshape(1, -1), W1, b1.reshape(1, -1),
      W2, b2.reshape(1, -1), g1.reshape(1, -1), be1.reshape(1, -1),
      g2.reshape(1, -1), be2.reshape(1, -1))


def kernel(src, spatial_shapes, level_start_index, pos,
           W_off, b_off, W_attn, b_attn, W_val, b_val, W_out, b_out,
           W1, b1, W2, b2, g1, be1, g2, be2):
    del spatial_shapes, level_start_index
    src2d = src.reshape(T, D_MODEL)
    pos2d = pos.reshape(T, D_MODEL)
    refp = jnp.asarray(_np_ref_points())
    refx = jnp.tile(refp[:, 0:1], (B, 1))
    refy = jnp.tile(refp[:, 1:2], (B, 1))
    W_off3 = W_off.reshape(D_MODEL, NCOL, 2)
    b_off2 = b_off.reshape(NCOL, 2)

    value, idx0, w4 = _run_proj(
        src2d, pos2d, refx, refy, W_val, b_val,
        W_off3[..., 0], W_off3[..., 1], b_off2[:, 0], b_off2[:, 1],
        W_attn, b_attn)

    table = _build_table(value)
    idx_flat = (idx0.reshape(B, LQ, NHEAD, NSAMP)
                .transpose(0, 2, 1, 3).reshape(-1))
    wts = jnp.stack([w4[0], w4[1], w4[2], w4[3]], axis=-1)
    wts_flat = (wts.reshape(B, LQ, NHEAD, NSAMP, 4)
                .transpose(0, 2, 1, 3, 4).reshape(-1))

    attn_flat = _run_sc(idx_flat, wts_flat, table)
    attn2d = (attn_flat.reshape(B, NHEAD, LQ, HD)
              .transpose(0, 2, 1, 3).reshape(T, D_MODEL))

    out = _run_tail(attn2d, src2d, W_out, b_out, W1, b1, W2, b2,
                    g1, be1, g2, be2)
    return out.reshape(B, LQ, D_MODEL)


# speedup vs baseline: 2156.6260x; 2.1878x over previous
import functools

import jax
import jax.numpy as jnp
import numpy as np
from jax import lax
from jax.experimental import pallas as pl
from jax.experimental.pallas import tpu as pltpu
from jax.experimental.pallas import tpu_sc as plsc

D_MODEL = 256
NHEAD = 8
HD = 32
NLEV = 4
NPOINT = 4
DFF = 2048
SHAPES_C = [(64, 64), (32, 32), (16, 16), (8, 8)]
LQ = sum(h * w for h, w in SHAPES_C)
B = 2
T = B * LQ
NSAMP = NLEV * NPOINT
NCOL = NHEAD * NSAMP

_bases = []
_off = 0
for _h, _w in SHAPES_C:
    _bases.append(_off)
    _off += (_h + 2) * (_w + 2)
BH_STRIDE = _off
N_BH = B * NHEAD
TABLE_ROWS = N_BH * BH_STRIDE

NW = 32
TOK_PER_W = T // NW

ROW_BLK = 544
N_BLK = T // ROW_BLK
BLK_PER_B = LQ // ROW_BLK


def _np_col_consts():
    w_col = np.zeros(NCOL, np.float32)
    h_col = np.zeros(NCOL, np.float32)
    base_col = np.zeros(NCOL, np.int32)
    head_col = np.zeros(NCOL, np.int32)
    for h in range(NHEAD):
        for l in range(NLEV):
            for p in range(NPOINT):
                c = h * NSAMP + l * NPOINT + p
                h_, w_ = SHAPES_C[l]
                w_col[c] = w_
                h_col[c] = h_
                base_col[c] = _bases[l]
                head_col[c] = h
    return w_col, h_col, base_col, head_col


def _np_ref_points():
    rps = []
    for (h_, w_) in SHAPES_C:
        ry, rx = np.meshgrid(np.linspace(0.5, h_ - 0.5, h_, dtype=np.float64),
                             np.linspace(0.5, w_ - 0.5, w_, dtype=np.float64),
                             indexing='ij')
        rps.append(np.stack([rx.reshape(-1) / w_, ry.reshape(-1) / h_], axis=-1))
    return np.concatenate(rps, axis=0).astype(np.float32)


_GROUP_ONES = np.kron(np.eye(NHEAD, dtype=np.float32),
                      np.ones((NSAMP, NSAMP), np.float32))


def _proj_body(src_ref, pos_ref, refx_ref, refy_ref,
               wval_ref, bval_ref, woffx_ref, woffy_ref, boffx_ref, boffy_ref,
               wattn_ref, battn_ref, gones_ref,
               wcol_ref, hcol_ref, basecol_ref, headcol_ref,
               value_ref, idx0_ref, w4_ref):
    x = src_ref[...]
    q = x + pos_ref[...]
    value_ref[...] = jnp.dot(x, wval_ref[...],
                             preferred_element_type=jnp.float32) + bval_ref[...]
    offx = jnp.dot(q, woffx_ref[...], preferred_element_type=jnp.float32) + boffx_ref[...]
    offy = jnp.dot(q, woffy_ref[...], preferred_element_type=jnp.float32) + boffy_ref[...]
    logits = jnp.dot(q, wattn_ref[...], preferred_element_type=jnp.float32) + battn_ref[...]
    m = jnp.max(logits, axis=-1, keepdims=True)
    e = jnp.exp(logits - m)
    denom = jnp.dot(e, gones_ref[...], preferred_element_type=jnp.float32)
    aw = e / denom

    wcol = wcol_ref[...]
    hcol = hcol_ref[...]
    gx = refx_ref[...] * wcol + offx - 0.5
    gy = refy_ref[...] * hcol + offy - 0.5
    x0 = jnp.floor(gx)
    y0 = jnp.floor(gy)
    wx1 = gx - x0
    wy1 = gy - y0
    wx0 = 1.0 - wx1
    wy0 = 1.0 - wy1
    xp = jnp.clip(x0, -1.0, wcol).astype(jnp.int32) + 1
    yr = jnp.clip(y0, -1.0, hcol).astype(jnp.int32) + 1
    msk = ((x0 >= -1.0) & (x0 <= wcol) & (y0 >= -1.0) & (y0 <= hcol)
           ).astype(jnp.float32)
    wp2 = wcol.astype(jnp.int32) + 2
    b = pl.program_id(0) // BLK_PER_B
    bh_base = (b * NHEAD + headcol_ref[...]) * BH_STRIDE + basecol_ref[...]
    idx0_ref[...] = bh_base + yr * wp2 + xp
    awm = aw * msk
    w4_ref[:, 0, :] = awm * wy0 * wx0
    w4_ref[:, 1, :] = awm * wy0 * wx1
    w4_ref[:, 2, :] = awm * wy1 * wx0
    w4_ref[:, 3, :] = awm * wy1 * wx1


def _run_proj(src2d, pos2d, refx, refy, W_val, b_val, W_offx, W_offy,
              b_offx, b_offy, W_attn, b_attn):
    wcol, hcol, basecol, headcol = _np_col_consts()
    full = lambda r, c: pl.BlockSpec((r, c), lambda i: (0, 0))
    blk = lambda c: pl.BlockSpec((ROW_BLK, c), lambda i: (i, 0))
    w4spec = pl.BlockSpec((ROW_BLK, 4, NCOL), lambda i: (i, 0, 0))
    return pl.pallas_call(
        _proj_body,
        grid=(N_BLK,),
        in_specs=[blk(D_MODEL), blk(D_MODEL), blk(1), blk(1),
                  full(D_MODEL, D_MODEL), full(1, D_MODEL),
                  full(D_MODEL, NCOL), full(D_MODEL, NCOL),
                  full(1, NCOL), full(1, NCOL),
                  full(D_MODEL, NCOL), full(1, NCOL), full(NCOL, NCOL),
                  full(1, NCOL), full(1, NCOL), full(1, NCOL), full(1, NCOL)],
        out_specs=[blk(D_MODEL), blk(NCOL), w4spec],
        out_shape=[jax.ShapeDtypeStruct((T, D_MODEL), jnp.float32),
                   jax.ShapeDtypeStruct((T, NCOL), jnp.int32),
                   jax.ShapeDtypeStruct((T, 4, NCOL), jnp.float32)],
    )(src2d, pos2d, refx, refy, W_val, b_val.reshape(1, -1),
      W_offx, W_offy, b_offx.reshape(1, -1), b_offy.reshape(1, -1),
      W_attn, b_attn.reshape(1, -1), jnp.asarray(_GROUP_ONES),
      jnp.asarray(wcol).reshape(1, -1), jnp.asarray(hcol).reshape(1, -1),
      jnp.asarray(basecol).reshape(1, -1), jnp.asarray(headcol).reshape(1, -1))


def _table_body(v_ref, out_ref):
    h = pl.program_id(0) % NHEAD
    v = jnp.zeros((LQ, HD), jnp.float32)
    for k in range(NHEAD):
        v = v + jnp.where(h == k, 1.0, 0.0) * v_ref[:, k * HD:(k + 1) * HD]
    start = 0
    for li, (h_, w_) in enumerate(SHAPES_C):
        hp, wp = h_ + 2, w_ + 2
        m = v[start:start + h_ * w_, :].reshape(h_, w_, HD)
        zrow = jnp.zeros((1, w_ + 2, HD), jnp.float32)
        zcol = jnp.zeros((h_, 1, HD), jnp.float32)
        mp = jnp.concatenate(
            [zrow, jnp.concatenate([zcol, m, zcol], axis=1), zrow], axis=0)
        zr = jnp.zeros((1, wp, HD), jnp.float32)
        zc = jnp.zeros((hp, 1, HD), jnp.float32)
        sx = jnp.concatenate([mp[:, 1:], zc], axis=1)
        sy = jnp.concatenate([mp[1:], zr], axis=0)
        sxy = jnp.concatenate([sx[1:], zr], axis=0)
        patch = jnp.concatenate([mp, sx, sy, sxy], axis=-1)
        out_ref[0, _bases[li]:_bases[li] + hp * wp, :] = patch.reshape(
            hp * wp, 4 * HD)
        start += h_ * w_


def _build_table(value):
    out = pl.pallas_call(
        _table_body,
        grid=(N_BH,),
        in_specs=[pl.BlockSpec((LQ, D_MODEL), lambda i: (i // NHEAD, 0))],
        out_specs=pl.BlockSpec((1, BH_STRIDE, 4 * HD), lambda i: (i, 0, 0)),
        out_shape=jax.ShapeDtypeStruct((N_BH, BH_STRIDE, 4 * HD), jnp.float32),
    )(value)
    return out.reshape(TABLE_ROWS, 4 * HD)


def _sc_body(idx_hbm, wts_hbm, table_hbm, out_hbm, idx_v, wts_v, rows_v, out_v, sem):
    wid = lax.axis_index("s") * 2 + lax.axis_index("c")
    t0 = wid * TOK_PER_W

    def tok(i, carry):
        t = t0 + i
        pltpu.sync_copy(idx_hbm.at[t], idx_v)
        pltpu.sync_copy(wts_hbm.at[t], wts_v)
        pltpu.async_copy(table_hbm.at[idx_v], rows_v, sem).wait()
        for h in range(NHEAD):
            acc0 = jnp.zeros((16,), jnp.float32)
            acc1 = jnp.zeros((16,), jnp.float32)
            wv0 = wts_v[0, pl.ds(h * NSAMP, 16)]
            wv1 = wts_v[1, pl.ds(h * NSAMP, 16)]
            wv2 = wts_v[2, pl.ds(h * NSAMP, 16)]
            wv3 = wts_v[3, pl.ds(h * NSAMP, 16)]
            for s in range(NSAMP):
                r = h * NSAMP + s
                w00 = jnp.full((16,), wv0[s], jnp.float32)
                w01 = jnp.full((16,), wv1[s], jnp.float32)
                w10 = jnp.full((16,), wv2[s], jnp.float32)
                w11 = jnp.full((16,), wv3[s], jnp.float32)
                acc0 = (acc0 + w00 * rows_v[r, 0:16] + w01 * rows_v[r, 32:48]
                        + w10 * rows_v[r, 64:80] + w11 * rows_v[r, 96:112])
                acc1 = (acc1 + w00 * rows_v[r, 16:32] + w01 * rows_v[r, 48:64]
                        + w10 * rows_v[r, 80:96] + w11 * rows_v[r, 112:128])
            out_v[pl.ds(h * HD, 16)] = acc0
            out_v[pl.ds(h * HD + 16, 16)] = acc1
        pltpu.sync_copy(out_v, out_hbm.at[t])
        return carry

    lax.fori_loop(0, TOK_PER_W, tok, 0)


def _run_sc(idx, wts4, table):
    mesh = plsc.VectorSubcoreMesh(core_axis_name="c", subcore_axis_name="s")
    kern = pl.kernel(
        _sc_body,
        out_type=jax.ShapeDtypeStruct((T, D_MODEL), jnp.float32),
        mesh=mesh,
        scratch_types=[
            pltpu.VMEM((NCOL,), jnp.int32),
            pltpu.VMEM((4, NCOL), jnp.float32),
            pltpu.VMEM((NCOL, 4 * HD), jnp.float32),
            pltpu.VMEM((D_MODEL,), jnp.float32),
            pltpu.SemaphoreType.DMA,
        ],
    )
    return kern(idx, wts4, table)


def _tail_body(attn_ref, src_ref, wout_ref, bout_ref, w1_ref, b1_ref,
               w2_ref, b2_ref, g1_ref, be1_ref, g2_ref, be2_ref, out_ref):
    def ln(x, g, bta):
        mu = jnp.mean(x, axis=-1, keepdims=True)
        xc = x - mu
        var = jnp.mean(xc * xc, axis=-1, keepdims=True)
        return xc * jax.lax.rsqrt(var + 1e-5) * g + bta

    src2 = jnp.dot(attn_ref[...], wout_ref[...],
                   preferred_element_type=jnp.float32) + bout_ref[...]
    x = ln(src_ref[...] + src2, g1_ref[...], be1_ref[...])
    h = jnp.maximum(jnp.dot(x, w1_ref[...], preferred_element_type=jnp.float32)
                    + b1_ref[...], 0.0)
    ff = jnp.dot(h, w2_ref[...], preferred_element_type=jnp.float32) + b2_ref[...]
    out_ref[...] = ln(x + ff, g2_ref[...], be2_ref[...])


def _run_tail(attn2d, src2d, W_out, b_out, W1, b1, W2, b2, g1, be1, g2, be2):
    full = lambda r, c: pl.BlockSpec((r, c), lambda i: (0, 0))
    blk = lambda c: pl.BlockSpec((ROW_BLK, c), lambda i: (i, 0))
    return pl.pallas_call(
        _tail_body,
        grid=(N_BLK,),
        in_specs=[blk(D_MODEL), blk(D_MODEL),
                  full(D_MODEL, D_MODEL), full(1, D_MODEL),
                  full(D_MODEL, DFF), full(1, DFF),
                  full(DFF, D_MODEL), full(1, D_MODEL),
                  full(1, D_MODEL), full(1, D_MODEL),
                  full(1, D_MODEL), full(1, D_MODEL)],
        out_specs=blk(D_MODEL),
        out_shape=jax.ShapeDtypeStruct((T, D_MODEL), jnp.float32),
    )(attn2d, src2d, W_out, b_out.reshape(1, -1), W1, b1.reshape(1, -1),
      W2, b2.reshape(1, -1), g1.reshape(1, -1), be1.reshape(1, -1),
      g2.reshape(1, -1), be2.reshape(1, -1))


def kernel(src, spatial_shapes, level_start_index, pos,
           W_off, b_off, W_attn, b_attn, W_val, b_val, W_out, b_out,
           W1, b1, W2, b2, g1, be1, g2, be2):
    del spatial_shapes, level_start_index
    src2d = src.reshape(T, D_MODEL)
    pos2d = pos.reshape(T, D_MODEL)
    refp = jnp.asarray(_np_ref_points())
    refx = jnp.tile(refp[:, 0:1], (B, 1))
    refy = jnp.tile(refp[:, 1:2], (B, 1))
    W_off3 = W_off.reshape(D_MODEL, NCOL, 2)
    b_off2 = b_off.reshape(NCOL, 2)

    value, idx0, w4 = _run_proj(
        src2d, pos2d, refx, refy, W_val, b_val,
        W_off3[..., 0], W_off3[..., 1], b_off2[:, 0], b_off2[:, 1],
        W_attn, b_attn)

    table = _build_table(value)
    attn2d = _run_sc(idx0, w4, table)

    out = _run_tail(attn2d, src2d, W_out, b_out, W1, b1, W2, b2,
                    g1, be1, g2, be2)
    return out.reshape(B, LQ, D_MODEL)


# speedup vs baseline: 3405.1973x; 1.5789x over previous
import functools

import jax
import jax.numpy as jnp
import numpy as np
from jax import lax
from jax.experimental import pallas as pl
from jax.experimental.pallas import tpu as pltpu
from jax.experimental.pallas import tpu_sc as plsc

D_MODEL = 256
NHEAD = 8
HD = 32
NLEV = 4
NPOINT = 4
DFF = 2048
SHAPES_C = [(64, 64), (32, 32), (16, 16), (8, 8)]
LQ = sum(h * w for h, w in SHAPES_C)
B = 2
T = B * LQ
NSAMP = NLEV * NPOINT
NCOL = NHEAD * NSAMP

_bases = []
_off = 0
for _h, _w in SHAPES_C:
    _bases.append(_off)
    _off += (_h + 2) * (_w + 2)
BH_STRIDE = _off
N_BH = B * NHEAD
TABLE_ROWS = N_BH * BH_STRIDE

NW = 32
TOK_PER_W = T // NW
CT = 2
NG = TOK_PER_W // CT

ROW_BLK = 544
N_BLK = T // ROW_BLK
BLK_PER_B = LQ // ROW_BLK


def _np_col_consts():
    w_col = np.zeros(NCOL, np.float32)
    h_col = np.zeros(NCOL, np.float32)
    base_col = np.zeros(NCOL, np.int32)
    head_col = np.zeros(NCOL, np.int32)
    for h in range(NHEAD):
        for l in range(NLEV):
            for p in range(NPOINT):
                c = h * NSAMP + l * NPOINT + p
                h_, w_ = SHAPES_C[l]
                w_col[c] = w_
                h_col[c] = h_
                base_col[c] = _bases[l]
                head_col[c] = h
    return w_col, h_col, base_col, head_col


def _np_ref_points():
    rps = []
    for (h_, w_) in SHAPES_C:
        ry, rx = np.meshgrid(np.linspace(0.5, h_ - 0.5, h_, dtype=np.float64),
                             np.linspace(0.5, w_ - 0.5, w_, dtype=np.float64),
                             indexing='ij')
        rps.append(np.stack([rx.reshape(-1) / w_, ry.reshape(-1) / h_], axis=-1))
    return np.concatenate(rps, axis=0).astype(np.float32)


_GROUP_ONES = np.kron(np.eye(NHEAD, dtype=np.float32),
                      np.ones((NSAMP, NSAMP), np.float32))


def _proj_body(src_ref, pos_ref, refx_ref, refy_ref,
               wval_ref, bval_ref, woffx_ref, woffy_ref, boffx_ref, boffy_ref,
               wattn_ref, battn_ref, gones_ref,
               wcol_ref, hcol_ref, basecol_ref, headcol_ref,
               value_ref, idx0_ref, w4_ref):
    x = src_ref[...]
    q = x + pos_ref[...]
    value_ref[...] = jnp.dot(x, wval_ref[...],
                             preferred_element_type=jnp.float32) + bval_ref[...]
    offx = jnp.dot(q, woffx_ref[...], preferred_element_type=jnp.float32) + boffx_ref[...]
    offy = jnp.dot(q, woffy_ref[...], preferred_element_type=jnp.float32) + boffy_ref[...]
    logits = jnp.dot(q, wattn_ref[...], preferred_element_type=jnp.float32) + battn_ref[...]
    m = jnp.max(logits, axis=-1, keepdims=True)
    e = jnp.exp(logits - m)
    denom = jnp.dot(e, gones_ref[...], preferred_element_type=jnp.float32)
    aw = e / denom

    wcol = wcol_ref[...]
    hcol = hcol_ref[...]
    gx = refx_ref[...] * wcol + offx - 0.5
    gy = refy_ref[...] * hcol + offy - 0.5
    x0 = jnp.floor(gx)
    y0 = jnp.floor(gy)
    wx1 = gx - x0
    wy1 = gy - y0
    wx0 = 1.0 - wx1
    wy0 = 1.0 - wy1
    xp = jnp.clip(x0, -1.0, wcol).astype(jnp.int32) + 1
    yr = jnp.clip(y0, -1.0, hcol).astype(jnp.int32) + 1
    msk = ((x0 >= -1.0) & (x0 <= wcol) & (y0 >= -1.0) & (y0 <= hcol)
           ).astype(jnp.float32)
    wp2 = wcol.astype(jnp.int32) + 2
    b = pl.program_id(0) // BLK_PER_B
    bh_base = (b * NHEAD + headcol_ref[...]) * BH_STRIDE + basecol_ref[...]
    idx0_ref[...] = bh_base + yr * wp2 + xp
    awm = aw * msk
    w4_ref[:, 0, :] = awm * wy0 * wx0
    w4_ref[:, 1, :] = awm * wy0 * wx1
    w4_ref[:, 2, :] = awm * wy1 * wx0
    w4_ref[:, 3, :] = awm * wy1 * wx1


def _run_proj(src2d, pos2d, refx, refy, W_val, b_val, W_offx, W_offy,
              b_offx, b_offy, W_attn, b_attn):
    wcol, hcol, basecol, headcol = _np_col_consts()
    full = lambda r, c: pl.BlockSpec((r, c), lambda i: (0, 0))
    blk = lambda c: pl.BlockSpec((ROW_BLK, c), lambda i: (i, 0))
    w4spec = pl.BlockSpec((ROW_BLK, 4, NCOL), lambda i: (i, 0, 0))
    return pl.pallas_call(
        _proj_body,
        grid=(N_BLK,),
        in_specs=[blk(D_MODEL), blk(D_MODEL), blk(1), blk(1),
                  full(D_MODEL, D_MODEL), full(1, D_MODEL),
                  full(D_MODEL, NCOL), full(D_MODEL, NCOL),
                  full(1, NCOL), full(1, NCOL),
                  full(D_MODEL, NCOL), full(1, NCOL), full(NCOL, NCOL),
                  full(1, NCOL), full(1, NCOL), full(1, NCOL), full(1, NCOL)],
        out_specs=[blk(D_MODEL), blk(NCOL), w4spec],
        out_shape=[jax.ShapeDtypeStruct((T, D_MODEL), jnp.float32),
                   jax.ShapeDtypeStruct((T, NCOL), jnp.int32),
                   jax.ShapeDtypeStruct((T, 4, NCOL), jnp.float32)],
    )(src2d, pos2d, refx, refy, W_val, b_val.reshape(1, -1),
      W_offx, W_offy, b_offx.reshape(1, -1), b_offy.reshape(1, -1),
      W_attn, b_attn.reshape(1, -1), jnp.asarray(_GROUP_ONES),
      jnp.asarray(wcol).reshape(1, -1), jnp.asarray(hcol).reshape(1, -1),
      jnp.asarray(basecol).reshape(1, -1), jnp.asarray(headcol).reshape(1, -1))


def _table_body(v_ref, out_ref):
    h = pl.program_id(0) % NHEAD
    v = jnp.zeros((LQ, HD), jnp.float32)
    for k in range(NHEAD):
        v = v + jnp.where(h == k, 1.0, 0.0) * v_ref[:, k * HD:(k + 1) * HD]
    start = 0
    for li, (h_, w_) in enumerate(SHAPES_C):
        hp, wp = h_ + 2, w_ + 2
        m = v[start:start + h_ * w_, :].reshape(h_, w_, HD)
        zrow = jnp.zeros((1, w_ + 2, HD), jnp.float32)
        zcol = jnp.zeros((h_, 1, HD), jnp.float32)
        mp = jnp.concatenate(
            [zrow, jnp.concatenate([zcol, m, zcol], axis=1), zrow], axis=0)
        zr = jnp.zeros((1, wp, HD), jnp.float32)
        zc = jnp.zeros((hp, 1, HD), jnp.float32)
        sx = jnp.concatenate([mp[:, 1:], zc], axis=1)
        sy = jnp.concatenate([mp[1:], zr], axis=0)
        sxy = jnp.concatenate([sx[1:], zr], axis=0)
        patch = jnp.concatenate([mp, sx, sy, sxy], axis=-1)
        out_ref[0, _bases[li]:_bases[li] + hp * wp, :] = patch.reshape(
            hp * wp, 4 * HD)
        start += h_ * w_


def _build_table(value):
    out = pl.pallas_call(
        _table_body,
        grid=(N_BH,),
        in_specs=[pl.BlockSpec((LQ, D_MODEL), lambda i: (i // NHEAD, 0))],
        out_specs=pl.BlockSpec((1, BH_STRIDE, 4 * HD), lambda i: (i, 0, 0)),
        out_shape=jax.ShapeDtypeStruct((N_BH, BH_STRIDE, 4 * HD), jnp.float32),
    )(value)
    return out.reshape(TABLE_ROWS, 4 * HD)


def _sc_body(idx_hbm, wts_hbm, table_hbm, out_hbm, idx_v, wts_v, rows_v, out_v, sem):
    wid = lax.axis_index("s") * 2 + lax.axis_index("c")
    t0 = wid * TOK_PER_W

    def fetch(g, buf):
        t = t0 + g * CT
        pltpu.sync_copy(idx_hbm.at[pl.ds(t, CT)], idx_v.at[buf])
        pltpu.sync_copy(wts_hbm.at[pl.ds(t, CT)], wts_v.at[buf])
        for j in range(CT):
            pltpu.async_copy(table_hbm.at[idx_v.at[buf, j]], rows_v.at[buf, j],
                             sem)

    fetch(0, 0)

    def chunk(g, carry):
        p = lax.rem(g, 2)

        @pl.when(g + 1 < NG)
        def _():
            fetch(g + 1, 1 - p)

        for j in range(CT):
            pltpu.make_async_copy(table_hbm.at[idx_v.at[p, j]],
                                  rows_v.at[p, j], sem).wait()
            for h in range(NHEAD):
                acc0 = jnp.zeros((16,), jnp.float32)
                acc1 = jnp.zeros((16,), jnp.float32)
                wv0 = wts_v[p, j, 0, pl.ds(h * NSAMP, 16)]
                wv1 = wts_v[p, j, 1, pl.ds(h * NSAMP, 16)]
                wv2 = wts_v[p, j, 2, pl.ds(h * NSAMP, 16)]
                wv3 = wts_v[p, j, 3, pl.ds(h * NSAMP, 16)]
                for s in range(NSAMP):
                    r = h * NSAMP + s
                    w00 = jnp.full((16,), wv0[s], jnp.float32)
                    w01 = jnp.full((16,), wv1[s], jnp.float32)
                    w10 = jnp.full((16,), wv2[s], jnp.float32)
                    w11 = jnp.full((16,), wv3[s], jnp.float32)
                    acc0 = (acc0 + w00 * rows_v[p, j, r, 0:16]
                            + w01 * rows_v[p, j, r, 32:48]
                            + w10 * rows_v[p, j, r, 64:80]
                            + w11 * rows_v[p, j, r, 96:112])
                    acc1 = (acc1 + w00 * rows_v[p, j, r, 16:32]
                            + w01 * rows_v[p, j, r, 48:64]
                            + w10 * rows_v[p, j, r, 80:96]
                            + w11 * rows_v[p, j, r, 112:128])
                out_v[j, pl.ds(h * HD, 16)] = acc0
                out_v[j, pl.ds(h * HD + 16, 16)] = acc1
        pltpu.sync_copy(out_v, out_hbm.at[pl.ds(t0 + g * CT, CT)])
        return carry

    lax.fori_loop(0, NG, chunk, 0)


def _run_sc(idx, wts4, table):
    mesh = plsc.VectorSubcoreMesh(core_axis_name="c", subcore_axis_name="s")
    kern = pl.kernel(
        _sc_body,
        out_type=jax.ShapeDtypeStruct((T, D_MODEL), jnp.float32),
        mesh=mesh,
        scratch_types=[
            pltpu.VMEM((2, CT, NCOL), jnp.int32),
            pltpu.VMEM((2, CT, 4, NCOL), jnp.float32),
            pltpu.VMEM((2, CT, NCOL, 4 * HD), jnp.float32),
            pltpu.VMEM((CT, D_MODEL), jnp.float32),
            pltpu.SemaphoreType.DMA,
        ],
    )
    return kern(idx, wts4, table)


def _tail_body(attn_ref, src_ref, wout_ref, bout_ref, w1_ref, b1_ref,
               w2_ref, b2_ref, g1_ref, be1_ref, g2_ref, be2_ref, out_ref):
    def ln(x, g, bta):
        mu = jnp.mean(x, axis=-1, keepdims=True)
        xc = x - mu
        var = jnp.mean(xc * xc, axis=-1, keepdims=True)
        return xc * jax.lax.rsqrt(var + 1e-5) * g + bta

    src2 = jnp.dot(attn_ref[...], wout_ref[...],
                   preferred_element_type=jnp.float32) + bout_ref[...]
    x = ln(src_ref[...] + src2, g1_ref[...], be1_ref[...])
    h = jnp.maximum(jnp.dot(x, w1_ref[...], preferred_element_type=jnp.float32)
                    + b1_ref[...], 0.0)
    ff = jnp.dot(h, w2_ref[...], preferred_element_type=jnp.float32) + b2_ref[...]
    out_ref[...] = ln(x + ff, g2_ref[...], be2_ref[...])


def _run_tail(attn2d, src2d, W_out, b_out, W1, b1, W2, b2, g1, be1, g2, be2):
    full = lambda r, c: pl.BlockSpec((r, c), lambda i: (0, 0))
    blk = lambda c: pl.BlockSpec((ROW_BLK, c), lambda i: (i, 0))
    return pl.pallas_call(
        _tail_body,
        grid=(N_BLK,),
        in_specs=[blk(D_MODEL), blk(D_MODEL),
                  full(D_MODEL, D_MODEL), full(1, D_MODEL),
                  full(D_MODEL, DFF), full(1, DFF),
                  full(DFF, D_MODEL), full(1, D_MODEL),
                  full(1, D_MODEL), full(1, D_MODEL),
                  full(1, D_MODEL), full(1, D_MODEL)],
        out_specs=blk(D_MODEL),
        out_shape=jax.ShapeDtypeStruct((T, D_MODEL), jnp.float32),
    )(attn2d, src2d, W_out, b_out.reshape(1, -1), W1, b1.reshape(1, -1),
      W2, b2.reshape(1, -1), g1.reshape(1, -1), be1.reshape(1, -1),
      g2.reshape(1, -1), be2.reshape(1, -1))


def kernel(src, spatial_shapes, level_start_index, pos,
           W_off, b_off, W_attn, b_attn, W_val, b_val, W_out, b_out,
           W1, b1, W2, b2, g1, be1, g2, be2):
    del spatial_shapes, level_start_index
    src2d = src.reshape(T, D_MODEL)
    pos2d = pos.reshape(T, D_MODEL)
    refp = jnp.asarray(_np_ref_points())
    refx = jnp.tile(refp[:, 0:1], (B, 1))
    refy = jnp.tile(refp[:, 1:2], (B, 1))
    W_off3 = W_off.reshape(D_MODEL, NCOL, 2)
    b_off2 = b_off.reshape(NCOL, 2)

    value, idx0, w4 = _run_proj(
        src2d, pos2d, refx, refy, W_val, b_val,
        W_off3[..., 0], W_off3[..., 1], b_off2[:, 0], b_off2[:, 1],
        W_attn, b_attn)

    table = _build_table(value)
    attn2d = _run_sc(idx0, w4, table)

    out = _run_tail(attn2d, src2d, W_out, b_out, W1, b1, W2, b2,
                    g1, be1, g2, be2)
    return out.reshape(B, LQ, D_MODEL)
